# baseline (speedup 1.0000x reference)
import functools

import jax
import jax.numpy as jnp
from jax import lax
from jax.experimental import pallas as pl
from jax.experimental.pallas import tpu as pltpu
from jax.experimental.pallas import tpu_sc as plsc

KNN = 16
RB = 80
CK = 256
NBATCH = 8
_BIG = 2 ** 30


def _knn_body(win_ref, brow_ref, bcol_ref, xblk_ref, xfull_ref, nbr_ref,
              d_s, sq_s, *, n, npad, din):
    i = pl.program_id(0)
    nck = npad // CK

    @pl.when(i == 0)
    def _():
        ones_row = jnp.ones((8, din), jnp.float32)

        def sqbody(c, _):
            xc = xfull_ref[pl.ds(c * CK, CK), :]
            sq = lax.dot_general(ones_row, xc * xc, (((1,), (1,)), ((), ())),
                                 precision=lax.Precision.HIGHEST,
                                 preferred_element_type=jnp.float32)
            sq_s[:, pl.ds(c * CK, CK)] = sq
            return 0

        lax.fori_loop(0, nck, sqbody, 0)

    lo_c = win_ref[i, 0]
    hi_c = win_ref[i, 1]
    r0 = i * RB

    xb = xblk_ref[...]
    sqr = jnp.sum(xb * xb, axis=1, keepdims=True)
    xbb = xb.astype(jnp.bfloat16)
    brow = brow_ref[...]

    def dbody(c, _):
        xc = xfull_ref[pl.ds(c * CK, CK), :]
        dot = lax.dot_general(xbb, xc.astype(jnp.bfloat16),
                              (((1,), (1,)), ((), ())),
                              preferred_element_type=jnp.float32)
        d = sqr - 2.0 * dot + sq_s[0:1, pl.ds(c * CK, CK)]
        bc = bcol_ref[0:1, pl.ds(c * CK, CK)]
        gidx = lax.broadcasted_iota(jnp.int32, (RB, CK), 1) + c * CK
        ridx = lax.broadcasted_iota(jnp.int32, (RB, CK), 0) + r0
        bad = jnp.logical_or(bc != brow, gidx == ridx)
        d_s[:, pl.ds(c * CK, CK)] = jnp.where(bad, jnp.inf, d)
        return 0

    lax.fori_loop(lo_c, hi_c, dbody, 0)

    last_v = jnp.full((RB, 1), -jnp.inf, jnp.float32)
    cols = []
    for _t in range(KNN):
        def ebody(c, carry, lv=last_v):
            mv, mi = carry
            d = d_s[:, pl.ds(c * CK, CK)]
            dd = jnp.where(d > lv, d, jnp.inf)
            cv = jnp.min(dd, axis=1, keepdims=True)
            gidx = lax.broadcasted_iota(jnp.int32, (RB, CK), 1) + c * CK
            ci = jnp.min(jnp.where(dd == cv, gidx, _BIG), axis=1,
                         keepdims=True)
            better = cv < mv
            return (jnp.where(better, cv, mv), jnp.where(better, ci, mi))

        mv, mi = lax.fori_loop(
            lo_c, hi_c, ebody,
            (jnp.full((RB, 1), jnp.inf, jnp.float32),
             jnp.full((RB, 1), _BIG, jnp.int32)))
        mi = jnp.minimum(mi, npad - 1)
        cols.append(mi)
        last_v = mv

    nbr_ref[...] = jnp.concatenate(cols, axis=1)


def _knn_topk(x, xpad, brow, bcol, win, n, npad, din):
    nblk = n // RB
    return pl.pallas_call(
        functools.partial(_knn_body, n=n, npad=npad, din=din),
        grid=(nblk,),
        in_specs=[
            pl.BlockSpec(memory_space=pltpu.SMEM),
            pl.BlockSpec((RB, 1), lambda i: (i, 0)),
            pl.BlockSpec((1, npad), lambda i: (0, 0)),
            pl.BlockSpec((RB, din), lambda i: (i, 0)),
            pl.BlockSpec((npad, din), lambda i: (0, 0)),
        ],
        out_specs=pl.BlockSpec((RB, KNN), lambda i: (i, 0)),
        out_shape=jax.ShapeDtypeStruct((n, KNN), jnp.int32),
        scratch_shapes=[
            pltpu.VMEM((RB, npad), jnp.float32),
            pltpu.VMEM((8, npad), jnp.float32),
        ],
    )(win, brow, bcol, x, xpad)


def _make_scatter_max(n, npad, din, qc):
    info = plsc.get_sparse_core_info()
    ncores = info.num_cores
    nsub = info.num_subcores
    nworkers = ncores * nsub
    cpt = din // nworkers
    nqc = n // qc

    mesh = plsc.VectorSubcoreMesh(core_axis_name="c", subcore_axis_name="s")

    scratch = [pltpu.VMEM((qc * KNN,), jnp.int32)]
    scratch += [pltpu.VMEM((n,), jnp.float32) for _ in range(cpt)]
    scratch += [pltpu.VMEM((npad,), jnp.float32) for _ in range(cpt)]

    @functools.partial(
        pl.kernel, mesh=mesh,
        out_type=jax.ShapeDtypeStruct((din, npad), jnp.float32),
        scratch_types=scratch,
        compiler_params=pltpu.CompilerParams(needs_layout_passes=False),
    )
    def sc_scatter(nbr_hbm, xt_hbm, out_hbm, nbr_v, *bufs):
        xv = bufs[:cpt]
        acc = bufs[cpt:]
        wid = lax.axis_index("s") * ncores + lax.axis_index("c")
        c0 = wid * cpt

        for r in range(cpt):
            pltpu.sync_copy(xt_hbm.at[c0 + r], xv[r])

        neg = jnp.full((16,), -jnp.inf, jnp.float32)

        def initbody(j, _):
            for r in range(cpt):
                acc[r][pl.ds(j * 16, 16)] = neg
            return 0

        lax.fori_loop(0, npad // 16, initbody, 0)

        def chunkbody(ch, _):
            pltpu.sync_copy(nbr_hbm.at[pl.ds(ch * qc * KNN, qc * KNN)],
                            nbr_v)

            def gbody(g, _):
                qb = g * 16
                xq = [xv[r][pl.ds(ch * qc + qb, 16)] for r in range(cpt)]
                for j in range(16):
                    idx = nbr_v[pl.ds((qb + j) * KNN, KNN)]
                    for r in range(cpt):
                        vals = jnp.full((16,), xq[r][j], jnp.float32)
                        cur = plsc.load_gather(acc[r], [idx])
                        plsc.store_scatter(acc[r], [idx],
                                           jnp.maximum(cur, vals))
                return 0

            lax.fori_loop(0, qc // 16, gbody, 0)
            return 0

        lax.fori_loop(0, nqc, chunkbody, 0)

        for r in range(cpt):
            pltpu.sync_copy(acc[r], out_hbm.at[c0 + r])

    return sc_scatter


def _lin_bn_body(pt_ref, w_ref, b_ref, g_ref, be_ref, y_ref, *, n, npad):
    nck = npad // CK
    w = w_ref[...]

    def p1(c, s):
        p = pt_ref[:, pl.ds(c * CK, CK)]
        p = jnp.where(p == -jnp.inf, 0.0, p)
        y = lax.dot_general(p, w, (((0,), (1,)), ((), ())),
                            preferred_element_type=jnp.float32)
        y = y + b_ref[...]
        y_ref[pl.ds(c * CK, CK), :] = y
        gidx = lax.broadcasted_iota(jnp.int32, (CK, 1), 0) + c * CK
        return s + jnp.sum(jnp.where(gidx < n, y, 0.0), axis=0,
                           keepdims=True)

    s = lax.fori_loop(0, nck, p1, jnp.zeros((1, 128), jnp.float32))
    mean = s * (1.0 / n)

    def p2(c, v):
        y = y_ref[pl.ds(c * CK, CK), :]
        dev = y - mean
        gidx = lax.broadcasted_iota(jnp.int32, (CK, 1), 0) + c * CK
        return v + jnp.sum(jnp.where(gidx < n, dev * dev, 0.0), axis=0,
                           keepdims=True)

    var = lax.fori_loop(0, nck, p2, jnp.zeros((1, 128), jnp.float32))
    var = var * (1.0 / n)
    scale = g_ref[...] * lax.rsqrt(var + 1e-5)
    shift = be_ref[...] - mean * scale

    def p3(c, _):
        y = y_ref[pl.ds(c * CK, CK), :]
        y_ref[pl.ds(c * CK, CK), :] = jnp.maximum(y * scale + shift, 0.0)
        return 0

    lax.fori_loop(0, nck, p3, 0)


def _lin_bn_relu(pool_t, w, b, gamma, beta, n, npad):
    return pl.pallas_call(
        functools.partial(_lin_bn_body, n=n, npad=npad),
        out_shape=jax.ShapeDtypeStruct((npad, 128), jnp.float32),
    )(pool_t, w, b, gamma, beta)


def kernel(x, batch, W, b, gamma, beta):
    n, din = x.shape
    batch = batch.astype(jnp.int32)
    npad = ((n + CK - 1) // CK) * CK
    nblk = n // RB

    bounds = jnp.searchsorted(batch, jnp.arange(NBATCH + 1, dtype=jnp.int32))
    r0s = jnp.arange(nblk, dtype=jnp.int32) * RB
    lo = bounds[batch[r0s]]
    hi = bounds[batch[r0s + RB - 1] + 1]
    win = jnp.stack([lo // CK, (hi + CK - 1) // CK], axis=1).astype(jnp.int32)

    xpad = jnp.pad(x, ((0, npad - n), (0, 0)))
    bcol = jnp.pad(batch, (0, npad - n), constant_values=-1).reshape(1, npad)
    brow = batch.reshape(n, 1)

    nbr = _knn_topk(x, xpad, brow, bcol, win, n, npad, din)

    xt = x.T
    pool_t = _make_scatter_max(n, npad, din, qc=2000)(nbr.reshape(-1), xt)

    y = _lin_bn_relu(pool_t, W, b.reshape(1, -1), gamma.reshape(1, -1),
                     beta.reshape(1, -1), n, npad)
    return y[:n]


# speedup vs baseline: 1.5678x; 1.5678x over previous
import functools

import jax
import jax.numpy as jnp
from jax import lax
from jax.experimental import pallas as pl
from jax.experimental.pallas import tpu as pltpu
from jax.experimental.pallas import tpu_sc as plsc

KNN = 16
RB = 80
CK = 512
NBATCH = 8
_BIG = 2 ** 30


def _knn_body(win_ref, brow_ref, bcol_ref, xblk_ref, xfull_ref, nbr_ref,
              d_s, sq_s, *, n, npad, din):
    i = pl.program_id(0)
    nck = npad // CK

    @pl.when(i == 0)
    def _():
        ones_row = jnp.ones((8, din), jnp.float32)

        def sqbody(c, _):
            xc = xfull_ref[pl.ds(c * CK, CK), :]
            sq = lax.dot_general(ones_row, xc * xc, (((1,), (1,)), ((), ())),
                                 precision=lax.Precision.HIGHEST,
                                 preferred_element_type=jnp.float32)
            sq_s[:, pl.ds(c * CK, CK)] = sq
            return 0

        lax.fori_loop(0, nck, sqbody, 0)

    lo_c = win_ref[i, 0]
    hi_c = win_ref[i, 1]
    r0 = i * RB

    xb = xblk_ref[...]
    sqr = jnp.sum(xb * xb, axis=1, keepdims=True)
    xbb = xb.astype(jnp.bfloat16)
    brow = brow_ref[...]

    def dbody(c, _):
        xc = xfull_ref[pl.ds(c * CK, CK), :]
        dot = lax.dot_general(xbb, xc.astype(jnp.bfloat16),
                              (((1,), (1,)), ((), ())),
                              preferred_element_type=jnp.float32)
        d = sqr - 2.0 * dot + sq_s[0:1, pl.ds(c * CK, CK)]
        bc = bcol_ref[0:1, pl.ds(c * CK, CK)]
        gidx = lax.broadcasted_iota(jnp.int32, (RB, CK), 1) + c * CK
        ridx = lax.broadcasted_iota(jnp.int32, (RB, CK), 0) + r0
        bad = jnp.logical_or(bc != brow, gidx == ridx)
        d_s[:, pl.ds(c * CK, CK)] = jnp.where(bad, jnp.inf, d)
        return 0

    lax.fori_loop(lo_c, hi_c, dbody, 0)

    last_v = jnp.full((RB, 1), -jnp.inf, jnp.float32)
    cols = []
    for _t in range(KNN):
        def ebody(c, carry, lv=last_v):
            mv, mi = carry
            d = d_s[:, pl.ds(c * CK, CK)]
            dd = jnp.where(d > lv, d, jnp.inf)
            cv = jnp.min(dd, axis=1, keepdims=True)
            gidx = lax.broadcasted_iota(jnp.int32, (RB, CK), 1) + c * CK
            ci = jnp.min(jnp.where(dd == cv, gidx, _BIG), axis=1,
                         keepdims=True)
            better = cv < mv
            return (jnp.where(better, cv, mv), jnp.where(better, ci, mi))

        mv, mi = lax.fori_loop(
            lo_c, hi_c, ebody,
            (jnp.full((RB, 1), jnp.inf, jnp.float32),
             jnp.full((RB, 1), _BIG, jnp.int32)))
        mi = jnp.minimum(mi, npad - 1)
        cols.append(mi)
        last_v = mv

    nbr_ref[...] = jnp.concatenate(cols, axis=1)


def _knn_topk(x, xpad, brow, bcol, win, n, npad, din):
    nblk = n // RB
    return pl.pallas_call(
        functools.partial(_knn_body, n=n, npad=npad, din=din),
        grid=(nblk,),
        in_specs=[
            pl.BlockSpec(memory_space=pltpu.SMEM),
            pl.BlockSpec((RB, 1), lambda i: (i, 0)),
            pl.BlockSpec((1, npad), lambda i: (0, 0)),
            pl.BlockSpec((RB, din), lambda i: (i, 0)),
            pl.BlockSpec((npad, din), lambda i: (0, 0)),
        ],
        out_specs=pl.BlockSpec((RB, KNN), lambda i: (i, 0)),
        out_shape=jax.ShapeDtypeStruct((n, KNN), jnp.int32),
        scratch_shapes=[
            pltpu.VMEM((RB, npad), jnp.float32),
            pltpu.VMEM((8, npad), jnp.float32),
        ],
    )(win, brow, bcol, x, xpad)


def _make_scatter_max(n, npad, din, qc):
    info = plsc.get_sparse_core_info()
    ncores = info.num_cores
    nsub = info.num_subcores
    nworkers = ncores * nsub
    cpt = din // nworkers
    nqc = n // qc

    mesh = plsc.VectorSubcoreMesh(core_axis_name="c", subcore_axis_name="s")

    scratch = [pltpu.VMEM((qc * KNN,), jnp.int32)]
    scratch += [pltpu.VMEM((n,), jnp.float32) for _ in range(cpt)]
    scratch += [pltpu.VMEM((npad,), jnp.float32) for _ in range(cpt)]

    @functools.partial(
        pl.kernel, mesh=mesh,
        out_type=jax.ShapeDtypeStruct((din, npad), jnp.float32),
        scratch_types=scratch,
        compiler_params=pltpu.CompilerParams(needs_layout_passes=False),
    )
    def sc_scatter(nbr_hbm, xt_hbm, out_hbm, nbr_v, *bufs):
        xv = bufs[:cpt]
        acc = bufs[cpt:]
        wid = lax.axis_index("s") * ncores + lax.axis_index("c")
        c0 = wid * cpt

        for r in range(cpt):
            pltpu.sync_copy(xt_hbm.at[c0 + r], xv[r])

        neg = jnp.full((16,), -jnp.inf, jnp.float32)

        def initbody(j, _):
            for r in range(cpt):
                acc[r][pl.ds(j * 16, 16)] = neg
            return 0

        lax.fori_loop(0, npad // 16, initbody, 0)

        def chunkbody(ch, _):
            pltpu.sync_copy(nbr_hbm.at[pl.ds(ch * qc * KNN, qc * KNN)],
                            nbr_v)

            def gbody(g, _):
                qb = g * 16
                xq = [xv[r][pl.ds(ch * qc + qb, 16)] for r in range(cpt)]
                for j in range(16):
                    idx = nbr_v[pl.ds((qb + j) * KNN, KNN)]
                    for r in range(cpt):
                        vals = jnp.full((16,), xq[r][j], jnp.float32)
                        cur = plsc.load_gather(acc[r], [idx])
                        plsc.store_scatter(acc[r], [idx],
                                           jnp.maximum(cur, vals))
                return 0

            lax.fori_loop(0, qc // 16, gbody, 0)
            return 0

        lax.fori_loop(0, nqc, chunkbody, 0)

        for r in range(cpt):
            pltpu.sync_copy(acc[r], out_hbm.at[c0 + r])

    return sc_scatter


def _lin_bn_body(pt_ref, w_ref, b_ref, g_ref, be_ref, y_ref, *, n, npad):
    nck = npad // CK
    w = w_ref[...]

    def p1(c, s):
        p = pt_ref[:, pl.ds(c * CK, CK)]
        p = jnp.where(p == -jnp.inf, 0.0, p)
        y = lax.dot_general(p, w, (((0,), (1,)), ((), ())),
                            preferred_element_type=jnp.float32)
        y = y + b_ref[...]
        y_ref[pl.ds(c * CK, CK), :] = y
        gidx = lax.broadcasted_iota(jnp.int32, (CK, 1), 0) + c * CK
        return s + jnp.sum(jnp.where(gidx < n, y, 0.0), axis=0,
                           keepdims=True)

    s = lax.fori_loop(0, nck, p1, jnp.zeros((1, 128), jnp.float32))
    mean = s * (1.0 / n)

    def p2(c, v):
        y = y_ref[pl.ds(c * CK, CK), :]
        dev = y - mean
        gidx = lax.broadcasted_iota(jnp.int32, (CK, 1), 0) + c * CK
        return v + jnp.sum(jnp.where(gidx < n, dev * dev, 0.0), axis=0,
                           keepdims=True)

    var = lax.fori_loop(0, nck, p2, jnp.zeros((1, 128), jnp.float32))
    var = var * (1.0 / n)
    scale = g_ref[...] * lax.rsqrt(var + 1e-5)
    shift = be_ref[...] - mean * scale

    def p3(c, _):
        y = y_ref[pl.ds(c * CK, CK), :]
        y_ref[pl.ds(c * CK, CK), :] = jnp.maximum(y * scale + shift, 0.0)
        return 0

    lax.fori_loop(0, nck, p3, 0)


def _lin_bn_relu(pool_t, w, b, gamma, beta, n, npad):
    return pl.pallas_call(
        functools.partial(_lin_bn_body, n=n, npad=npad),
        out_shape=jax.ShapeDtypeStruct((npad, 128), jnp.float32),
    )(pool_t, w, b, gamma, beta)


def kernel(x, batch, W, b, gamma, beta):
    n, din = x.shape
    batch = batch.astype(jnp.int32)
    npad = ((n + CK - 1) // CK) * CK
    nblk = n // RB

    bounds = jnp.searchsorted(batch, jnp.arange(NBATCH + 1, dtype=jnp.int32))
    r0s = jnp.arange(nblk, dtype=jnp.int32) * RB
    lo = bounds[batch[r0s]]
    hi = bounds[batch[r0s + RB - 1] + 1]
    win = jnp.stack([lo // CK, (hi + CK - 1) // CK], axis=1).astype(jnp.int32)

    xpad = jnp.pad(x, ((0, npad - n), (0, 0)))
    bcol = jnp.pad(batch, (0, npad - n), constant_values=-1).reshape(1, npad)
    brow = batch.reshape(n, 1)

    nbr = _knn_topk(x, xpad, brow, bcol, win, n, npad, din)

    xt = x.T
    pool_t = _make_scatter_max(n, npad, din, qc=2000)(nbr.reshape(-1), xt)

    y = _lin_bn_relu(pool_t, W, b.reshape(1, -1), gamma.reshape(1, -1),
                     beta.reshape(1, -1), n, npad)
    return y[:n]


# speedup vs baseline: 2.1026x; 1.3411x over previous
import functools

import jax
import jax.numpy as jnp
from jax import lax
from jax.experimental import pallas as pl
from jax.experimental.pallas import tpu as pltpu
from jax.experimental.pallas import tpu_sc as plsc

KNN = 16
RB = 80
CK = 1024
NBATCH = 8
_BIG = 2 ** 30


def _knn_body(win_ref, brow_ref, bcol_ref, xblk_ref, xfull_ref, nbr_ref,
              d_s, sq_s, *, n, npad, din):
    i = pl.program_id(0)
    nck = npad // CK

    @pl.when(i == 0)
    def _():
        ones_row = jnp.ones((8, din), jnp.float32)

        def sqbody(c, _):
            xc = xfull_ref[pl.ds(c * CK, CK), :]
            sq = lax.dot_general(ones_row, xc * xc, (((1,), (1,)), ((), ())),
                                 precision=lax.Precision.HIGHEST,
                                 preferred_element_type=jnp.float32)
            sq_s[:, pl.ds(c * CK, CK)] = sq
            return 0

        lax.fori_loop(0, nck, sqbody, 0)

    lo_c = win_ref[i, 0]
    hi_c = win_ref[i, 1]
    r0 = i * RB

    xb = xblk_ref[...]
    sqr = jnp.sum(xb * xb, axis=1, keepdims=True)
    xbb = xb.astype(jnp.bfloat16)
    brow = brow_ref[...]

    def dbody(c, _):
        xc = xfull_ref[pl.ds(c * CK, CK), :]
        dot = lax.dot_general(xbb, xc.astype(jnp.bfloat16),
                              (((1,), (1,)), ((), ())),
                              preferred_element_type=jnp.float32)
        d = sqr - 2.0 * dot + sq_s[0:1, pl.ds(c * CK, CK)]
        bc = bcol_ref[0:1, pl.ds(c * CK, CK)]
        gidx = lax.broadcasted_iota(jnp.int32, (RB, CK), 1) + c * CK
        ridx = lax.broadcasted_iota(jnp.int32, (RB, CK), 0) + r0
        bad = jnp.logical_or(bc != brow, gidx == ridx)
        d_s[:, pl.ds(c * CK, CK)] = jnp.where(bad, jnp.inf, d)
        return 0

    lax.fori_loop(lo_c, hi_c, dbody, 0)

    last_v = jnp.full((RB, 1), -jnp.inf, jnp.float32)
    cols = []
    for _t in range(KNN):
        def ebody(c, carry, lv=last_v):
            mv, mi = carry
            d = d_s[:, pl.ds(c * CK, CK)]
            dd = jnp.where(d > lv, d, jnp.inf)
            cv = jnp.min(dd, axis=1, keepdims=True)
            gidx = lax.broadcasted_iota(jnp.int32, (RB, CK), 1) + c * CK
            ci = jnp.min(jnp.where(dd == cv, gidx, _BIG), axis=1,
                         keepdims=True)
            better = cv < mv
            return (jnp.where(better, cv, mv), jnp.where(better, ci, mi))

        mv, mi = lax.fori_loop(
            lo_c, hi_c, ebody,
            (jnp.full((RB, 1), jnp.inf, jnp.float32),
             jnp.full((RB, 1), _BIG, jnp.int32)))
        mi = jnp.minimum(mi, npad - 1)
        cols.append(mi)
        last_v = mv

    nbr_ref[...] = jnp.concatenate(cols, axis=1)


def _knn_topk(x, xpad, brow, bcol, win, n, npad, din):
    nblk = n // RB
    return pl.pallas_call(
        functools.partial(_knn_body, n=n, npad=npad, din=din),
        grid=(nblk,),
        in_specs=[
            pl.BlockSpec(memory_space=pltpu.SMEM),
            pl.BlockSpec((RB, 1), lambda i: (i, 0)),
            pl.BlockSpec((1, npad), lambda i: (0, 0)),
            pl.BlockSpec((RB, din), lambda i: (i, 0)),
            pl.BlockSpec((npad, din), lambda i: (0, 0)),
        ],
        out_specs=pl.BlockSpec((RB, KNN), lambda i: (i, 0)),
        out_shape=jax.ShapeDtypeStruct((n, KNN), jnp.int32),
        scratch_shapes=[
            pltpu.VMEM((RB, npad), jnp.float32),
            pltpu.VMEM((8, npad), jnp.float32),
        ],
    )(win, brow, bcol, x, xpad)


def _make_scatter_max(n, npad, din, qc):
    info = plsc.get_sparse_core_info()
    ncores = info.num_cores
    nsub = info.num_subcores
    nworkers = ncores * nsub
    cpt = din // nworkers
    nqc = n // qc

    mesh = plsc.VectorSubcoreMesh(core_axis_name="c", subcore_axis_name="s")

    scratch = [pltpu.VMEM((qc * KNN,), jnp.int32)]
    scratch += [pltpu.VMEM((n,), jnp.float32) for _ in range(cpt)]
    scratch += [pltpu.VMEM((npad,), jnp.float32) for _ in range(cpt)]

    @functools.partial(
        pl.kernel, mesh=mesh,
        out_type=jax.ShapeDtypeStruct((din, npad), jnp.float32),
        scratch_types=scratch,
        compiler_params=pltpu.CompilerParams(needs_layout_passes=False),
    )
    def sc_scatter(nbr_hbm, xt_hbm, out_hbm, nbr_v, *bufs):
        xv = bufs[:cpt]
        acc = bufs[cpt:]
        wid = lax.axis_index("s") * ncores + lax.axis_index("c")
        c0 = wid * cpt

        for r in range(cpt):
            pltpu.sync_copy(xt_hbm.at[c0 + r], xv[r])

        neg = jnp.full((16,), -jnp.inf, jnp.float32)

        def initbody(j, _):
            for r in range(cpt):
                acc[r][pl.ds(j * 16, 16)] = neg
            return 0

        lax.fori_loop(0, npad // 16, initbody, 0)

        def chunkbody(ch, _):
            pltpu.sync_copy(nbr_hbm.at[pl.ds(ch * qc * KNN, qc * KNN)],
                            nbr_v)

            def gbody(g, _):
                qb = g * 16
                xq = [xv[r][pl.ds(ch * qc + qb, 16)] for r in range(cpt)]
                for j in range(16):
                    idx = nbr_v[pl.ds((qb + j) * KNN, KNN)]
                    for r in range(cpt):
                        vals = jnp.full((16,), xq[r][j], jnp.float32)
                        cur = plsc.load_gather(acc[r], [idx])
                        plsc.store_scatter(acc[r], [idx],
                                           jnp.maximum(cur, vals))
                return 0

            lax.fori_loop(0, qc // 16, gbody, 0)
            return 0

        lax.fori_loop(0, nqc, chunkbody, 0)

        for r in range(cpt):
            pltpu.sync_copy(acc[r], out_hbm.at[c0 + r])

    return sc_scatter


def _lin_bn_body(pt_ref, w_ref, b_ref, g_ref, be_ref, y_ref, *, n, npad):
    nck = npad // CK
    w = w_ref[...]

    def p1(c, s):
        p = pt_ref[:, pl.ds(c * CK, CK)]
        p = jnp.where(p == -jnp.inf, 0.0, p)
        y = lax.dot_general(p, w, (((0,), (1,)), ((), ())),
                            preferred_element_type=jnp.float32)
        y = y + b_ref[...]
        y_ref[pl.ds(c * CK, CK), :] = y
        gidx = lax.broadcasted_iota(jnp.int32, (CK, 1), 0) + c * CK
        return s + jnp.sum(jnp.where(gidx < n, y, 0.0), axis=0,
                           keepdims=True)

    s = lax.fori_loop(0, nck, p1, jnp.zeros((1, 128), jnp.float32))
    mean = s * (1.0 / n)

    def p2(c, v):
        y = y_ref[pl.ds(c * CK, CK), :]
        dev = y - mean
        gidx = lax.broadcasted_iota(jnp.int32, (CK, 1), 0) + c * CK
        return v + jnp.sum(jnp.where(gidx < n, dev * dev, 0.0), axis=0,
                           keepdims=True)

    var = lax.fori_loop(0, nck, p2, jnp.zeros((1, 128), jnp.float32))
    var = var * (1.0 / n)
    scale = g_ref[...] * lax.rsqrt(var + 1e-5)
    shift = be_ref[...] - mean * scale

    def p3(c, _):
        y = y_ref[pl.ds(c * CK, CK), :]
        y_ref[pl.ds(c * CK, CK), :] = jnp.maximum(y * scale + shift, 0.0)
        return 0

    lax.fori_loop(0, nck, p3, 0)


def _lin_bn_relu(pool_t, w, b, gamma, beta, n, npad):
    return pl.pallas_call(
        functools.partial(_lin_bn_body, n=n, npad=npad),
        out_shape=jax.ShapeDtypeStruct((npad, 128), jnp.float32),
    )(pool_t, w, b, gamma, beta)


def kernel(x, batch, W, b, gamma, beta):
    n, din = x.shape
    batch = batch.astype(jnp.int32)
    npad = ((n + CK - 1) // CK) * CK
    nblk = n // RB

    bounds = jnp.searchsorted(batch, jnp.arange(NBATCH + 1, dtype=jnp.int32))
    r0s = jnp.arange(nblk, dtype=jnp.int32) * RB
    lo = bounds[batch[r0s]]
    hi = bounds[batch[r0s + RB - 1] + 1]
    win = jnp.stack([lo // CK, (hi + CK - 1) // CK], axis=1).astype(jnp.int32)

    xpad = jnp.pad(x, ((0, npad - n), (0, 0)))
    bcol = jnp.pad(batch, (0, npad - n), constant_values=-1).reshape(1, npad)
    brow = batch.reshape(n, 1)

    nbr = _knn_topk(x, xpad, brow, bcol, win, n, npad, din)

    xt = x.T
    pool_t = _make_scatter_max(n, npad, din, qc=2000)(nbr.reshape(-1), xt)

    y = _lin_bn_relu(pool_t, W, b.reshape(1, -1), gamma.reshape(1, -1),
                     beta.reshape(1, -1), n, npad)
    return y[:n]


# speedup vs baseline: 2.3829x; 1.1333x over previous
import functools

import jax
import jax.numpy as jnp
from jax import lax
from jax.experimental import pallas as pl
from jax.experimental.pallas import tpu as pltpu
from jax.experimental.pallas import tpu_sc as plsc

KNN = 16
RB = 80
CK = 2048
NBATCH = 8
_BIG = 2 ** 30


def _knn_body(win_ref, brow_ref, bcol_ref, xblk_ref, xfull_ref, nbr_ref,
              d_s, sq_s, *, n, npad, din):
    i = pl.program_id(0)
    nck = npad // CK

    @pl.when(i == 0)
    def _():
        ones_row = jnp.ones((8, din), jnp.float32)

        def sqbody(c, _):
            xc = xfull_ref[pl.ds(c * CK, CK), :]
            sq = lax.dot_general(ones_row, xc * xc, (((1,), (1,)), ((), ())),
                                 precision=lax.Precision.HIGHEST,
                                 preferred_element_type=jnp.float32)
            sq_s[:, pl.ds(c * CK, CK)] = sq
            return 0

        lax.fori_loop(0, nck, sqbody, 0)

    lo_c = win_ref[i, 0]
    hi_c = win_ref[i, 1]
    r0 = i * RB

    xb = xblk_ref[...]
    sqr = jnp.sum(xb * xb, axis=1, keepdims=True)
    xbb = xb.astype(jnp.bfloat16)
    brow = brow_ref[...]

    def dbody(c, _):
        xc = xfull_ref[pl.ds(c * CK, CK), :]
        dot = lax.dot_general(xbb, xc.astype(jnp.bfloat16),
                              (((1,), (1,)), ((), ())),
                              preferred_element_type=jnp.float32)
        d = sqr - 2.0 * dot + sq_s[0:1, pl.ds(c * CK, CK)]
        bc = bcol_ref[0:1, pl.ds(c * CK, CK)]
        gidx = lax.broadcasted_iota(jnp.int32, (RB, CK), 1) + c * CK
        ridx = lax.broadcasted_iota(jnp.int32, (RB, CK), 0) + r0
        bad = jnp.logical_or(bc != brow, gidx == ridx)
        d_s[:, pl.ds(c * CK, CK)] = jnp.where(bad, jnp.inf, d)
        return 0

    lax.fori_loop(lo_c, hi_c, dbody, 0)

    last_v = jnp.full((RB, 1), -jnp.inf, jnp.float32)
    cols = []
    for _t in range(KNN):
        def ebody(c, carry, lv=last_v):
            mv, mi = carry
            d = d_s[:, pl.ds(c * CK, CK)]
            dd = jnp.where(d > lv, d, jnp.inf)
            cv = jnp.min(dd, axis=1, keepdims=True)
            gidx = lax.broadcasted_iota(jnp.int32, (RB, CK), 1) + c * CK
            ci = jnp.min(jnp.where(dd == cv, gidx, _BIG), axis=1,
                         keepdims=True)
            better = cv < mv
            return (jnp.where(better, cv, mv), jnp.where(better, ci, mi))

        mv, mi = lax.fori_loop(
            lo_c, hi_c, ebody,
            (jnp.full((RB, 1), jnp.inf, jnp.float32),
             jnp.full((RB, 1), _BIG, jnp.int32)))
        mi = jnp.minimum(mi, npad - 1)
        cols.append(mi)
        last_v = mv

    nbr_ref[...] = jnp.concatenate(cols, axis=1)


def _knn_topk(x, xpad, brow, bcol, win, n, npad, din):
    nblk = n // RB
    return pl.pallas_call(
        functools.partial(_knn_body, n=n, npad=npad, din=din),
        grid=(nblk,),
        in_specs=[
            pl.BlockSpec(memory_space=pltpu.SMEM),
            pl.BlockSpec((RB, 1), lambda i: (i, 0)),
            pl.BlockSpec((1, npad), lambda i: (0, 0)),
            pl.BlockSpec((RB, din), lambda i: (i, 0)),
            pl.BlockSpec((npad, din), lambda i: (0, 0)),
        ],
        out_specs=pl.BlockSpec((RB, KNN), lambda i: (i, 0)),
        out_shape=jax.ShapeDtypeStruct((n, KNN), jnp.int32),
        scratch_shapes=[
            pltpu.VMEM((RB, npad), jnp.float32),
            pltpu.VMEM((8, npad), jnp.float32),
        ],
    )(win, brow, bcol, x, xpad)


def _make_scatter_max(n, npad, din, qc):
    info = plsc.get_sparse_core_info()
    ncores = info.num_cores
    nsub = info.num_subcores
    nworkers = ncores * nsub
    cpt = din // nworkers
    nqc = n // qc

    mesh = plsc.VectorSubcoreMesh(core_axis_name="c", subcore_axis_name="s")

    scratch = [pltpu.VMEM((qc * KNN,), jnp.int32)]
    scratch += [pltpu.VMEM((n,), jnp.float32) for _ in range(cpt)]
    scratch += [pltpu.VMEM((npad,), jnp.float32) for _ in range(cpt)]

    @functools.partial(
        pl.kernel, mesh=mesh,
        out_type=jax.ShapeDtypeStruct((din, npad), jnp.float32),
        scratch_types=scratch,
        compiler_params=pltpu.CompilerParams(needs_layout_passes=False),
    )
    def sc_scatter(nbr_hbm, xt_hbm, out_hbm, nbr_v, *bufs):
        xv = bufs[:cpt]
        acc = bufs[cpt:]
        wid = lax.axis_index("s") * ncores + lax.axis_index("c")
        c0 = wid * cpt

        for r in range(cpt):
            pltpu.sync_copy(xt_hbm.at[c0 + r], xv[r])

        neg = jnp.full((16,), -jnp.inf, jnp.float32)

        def initbody(j, _):
            for r in range(cpt):
                acc[r][pl.ds(j * 16, 16)] = neg
            return 0

        lax.fori_loop(0, npad // 16, initbody, 0)

        def chunkbody(ch, _):
            pltpu.sync_copy(nbr_hbm.at[pl.ds(ch * qc * KNN, qc * KNN)],
                            nbr_v)

            def gbody(g, _):
                qb = g * 16
                xq = [xv[r][pl.ds(ch * qc + qb, 16)] for r in range(cpt)]
                for j in range(16):
                    idx = nbr_v[pl.ds((qb + j) * KNN, KNN)]
                    for r in range(cpt):
                        vals = jnp.full((16,), xq[r][j], jnp.float32)
                        cur = plsc.load_gather(acc[r], [idx])
                        plsc.store_scatter(acc[r], [idx],
                                           jnp.maximum(cur, vals))
                return 0

            lax.fori_loop(0, qc // 16, gbody, 0)
            return 0

        lax.fori_loop(0, nqc, chunkbody, 0)

        for r in range(cpt):
            pltpu.sync_copy(acc[r], out_hbm.at[c0 + r])

    return sc_scatter


def _lin_bn_body(pt_ref, w_ref, b_ref, g_ref, be_ref, y_ref, *, n, npad):
    nck = npad // CK
    w = w_ref[...]

    def p1(c, s):
        p = pt_ref[:, pl.ds(c * CK, CK)]
        p = jnp.where(p == -jnp.inf, 0.0, p)
        y = lax.dot_general(p, w, (((0,), (1,)), ((), ())),
                            preferred_element_type=jnp.float32)
        y = y + b_ref[...]
        y_ref[pl.ds(c * CK, CK), :] = y
        gidx = lax.broadcasted_iota(jnp.int32, (CK, 1), 0) + c * CK
        return s + jnp.sum(jnp.where(gidx < n, y, 0.0), axis=0,
                           keepdims=True)

    s = lax.fori_loop(0, nck, p1, jnp.zeros((1, 128), jnp.float32))
    mean = s * (1.0 / n)

    def p2(c, v):
        y = y_ref[pl.ds(c * CK, CK), :]
        dev = y - mean
        gidx = lax.broadcasted_iota(jnp.int32, (CK, 1), 0) + c * CK
        return v + jnp.sum(jnp.where(gidx < n, dev * dev, 0.0), axis=0,
                           keepdims=True)

    var = lax.fori_loop(0, nck, p2, jnp.zeros((1, 128), jnp.float32))
    var = var * (1.0 / n)
    scale = g_ref[...] * lax.rsqrt(var + 1e-5)
    shift = be_ref[...] - mean * scale

    def p3(c, _):
        y = y_ref[pl.ds(c * CK, CK), :]
        y_ref[pl.ds(c * CK, CK), :] = jnp.maximum(y * scale + shift, 0.0)
        return 0

    lax.fori_loop(0, nck, p3, 0)


def _lin_bn_relu(pool_t, w, b, gamma, beta, n, npad):
    return pl.pallas_call(
        functools.partial(_lin_bn_body, n=n, npad=npad),
        out_shape=jax.ShapeDtypeStruct((npad, 128), jnp.float32),
    )(pool_t, w, b, gamma, beta)


def kernel(x, batch, W, b, gamma, beta):
    n, din = x.shape
    batch = batch.astype(jnp.int32)
    npad = ((n + CK - 1) // CK) * CK
    nblk = n // RB

    bounds = jnp.searchsorted(batch, jnp.arange(NBATCH + 1, dtype=jnp.int32))
    r0s = jnp.arange(nblk, dtype=jnp.int32) * RB
    lo = bounds[batch[r0s]]
    hi = bounds[batch[r0s + RB - 1] + 1]
    win = jnp.stack([lo // CK, (hi + CK - 1) // CK], axis=1).astype(jnp.int32)

    xpad = jnp.pad(x, ((0, npad - n), (0, 0)))
    bcol = jnp.pad(batch, (0, npad - n), constant_values=-1).reshape(1, npad)
    brow = batch.reshape(n, 1)

    nbr = _knn_topk(x, xpad, brow, bcol, win, n, npad, din)

    xt = x.T
    pool_t = _make_scatter_max(n, npad, din, qc=2000)(nbr.reshape(-1), xt)

    y = _lin_bn_relu(pool_t, W, b.reshape(1, -1), gamma.reshape(1, -1),
                     beta.reshape(1, -1), n, npad)
    return y[:n]


# speedup vs baseline: 3.4651x; 1.4542x over previous
import functools

import jax
import jax.numpy as jnp
from jax import lax
from jax.experimental import pallas as pl
from jax.experimental.pallas import tpu as pltpu
from jax.experimental.pallas import tpu_sc as plsc

KNN = 16
RB = 400
CK = 2048
NBATCH = 8
_BIG = 2 ** 30


def _knn_body(win_ref, brow_ref, bcol_ref, xblk_ref, xfull_ref, nbr_ref,
              d_s, sq_s, *, n, npad, din):
    i = pl.program_id(0)
    nck = npad // CK

    @pl.when(i == 0)
    def _():
        ones_row = jnp.ones((8, din), jnp.float32)

        def sqbody(c, _):
            xc = xfull_ref[pl.ds(c * CK, CK), :]
            sq = lax.dot_general(ones_row, xc * xc, (((1,), (1,)), ((), ())),
                                 precision=lax.Precision.HIGHEST,
                                 preferred_element_type=jnp.float32)
            sq_s[:, pl.ds(c * CK, CK)] = sq
            return 0

        lax.fori_loop(0, nck, sqbody, 0)

    lo_c = win_ref[i, 0]
    hi_c = win_ref[i, 1]
    r0 = i * RB

    xb = xblk_ref[...]
    sqr = jnp.sum(xb * xb, axis=1, keepdims=True)
    xbb = xb.astype(jnp.bfloat16)
    brow = brow_ref[...]

    def dbody(c, _):
        xc = xfull_ref[pl.ds(c * CK, CK), :]
        dot = lax.dot_general(xbb, xc.astype(jnp.bfloat16),
                              (((1,), (1,)), ((), ())),
                              preferred_element_type=jnp.float32)
        d = sqr - 2.0 * dot + sq_s[0:1, pl.ds(c * CK, CK)]
        bc = bcol_ref[0:1, pl.ds(c * CK, CK)]
        gidx = lax.broadcasted_iota(jnp.int32, (RB, CK), 1) + c * CK
        ridx = lax.broadcasted_iota(jnp.int32, (RB, CK), 0) + r0
        bad = jnp.logical_or(bc != brow, gidx == ridx)
        d_s[:, pl.ds(c * CK, CK)] = jnp.where(bad, jnp.inf, d)
        return 0

    lax.fori_loop(lo_c, hi_c, dbody, 0)

    last_v = jnp.full((RB, 1), -jnp.inf, jnp.float32)
    cols = []
    for _t in range(KNN):
        def ebody(c, carry, lv=last_v):
            mv, mi = carry
            d = d_s[:, pl.ds(c * CK, CK)]
            dd = jnp.where(d > lv, d, jnp.inf)
            cv = jnp.min(dd, axis=1, keepdims=True)
            gidx = lax.broadcasted_iota(jnp.int32, (RB, CK), 1) + c * CK
            ci = jnp.min(jnp.where(dd == cv, gidx, _BIG), axis=1,
                         keepdims=True)
            better = cv < mv
            return (jnp.where(better, cv, mv), jnp.where(better, ci, mi))

        mv, mi = lax.fori_loop(
            lo_c, hi_c, ebody,
            (jnp.full((RB, 1), jnp.inf, jnp.float32),
             jnp.full((RB, 1), _BIG, jnp.int32)))
        mi = jnp.minimum(mi, npad - 1)
        cols.append(mi)
        last_v = mv

    nbr_ref[...] = jnp.concatenate(cols, axis=1)


def _knn_topk(x, xpad, brow, bcol, win, n, npad, din):
    nblk = n // RB
    return pl.pallas_call(
        functools.partial(_knn_body, n=n, npad=npad, din=din),
        grid=(nblk,),
        in_specs=[
            pl.BlockSpec(memory_space=pltpu.SMEM),
            pl.BlockSpec((RB, 1), lambda i: (i, 0)),
            pl.BlockSpec((1, npad), lambda i: (0, 0)),
            pl.BlockSpec((RB, din), lambda i: (i, 0)),
            pl.BlockSpec((npad, din), lambda i: (0, 0)),
        ],
        out_specs=pl.BlockSpec((RB, KNN), lambda i: (i, 0)),
        out_shape=jax.ShapeDtypeStruct((n, KNN), jnp.int32),
        scratch_shapes=[
            pltpu.VMEM((RB, npad), jnp.float32),
            pltpu.VMEM((8, npad), jnp.float32),
        ],
    )(win, brow, bcol, x, xpad)


def _make_scatter_max(n, npad, din, qc):
    info = plsc.get_sparse_core_info()
    ncores = info.num_cores
    nsub = info.num_subcores
    nworkers = ncores * nsub
    cpt = din // nworkers
    nqc = n // qc

    mesh = plsc.VectorSubcoreMesh(core_axis_name="c", subcore_axis_name="s")

    scratch = [pltpu.VMEM((qc * KNN,), jnp.int32)]
    scratch += [pltpu.VMEM((n,), jnp.float32) for _ in range(cpt)]
    scratch += [pltpu.VMEM((npad,), jnp.float32) for _ in range(cpt)]

    @functools.partial(
        pl.kernel, mesh=mesh,
        out_type=jax.ShapeDtypeStruct((din, npad), jnp.float32),
        scratch_types=scratch,
        compiler_params=pltpu.CompilerParams(needs_layout_passes=False),
    )
    def sc_scatter(nbr_hbm, xt_hbm, out_hbm, nbr_v, *bufs):
        xv = bufs[:cpt]
        acc = bufs[cpt:]
        wid = lax.axis_index("s") * ncores + lax.axis_index("c")
        c0 = wid * cpt

        for r in range(cpt):
            pltpu.sync_copy(xt_hbm.at[c0 + r], xv[r])

        neg = jnp.full((16,), -jnp.inf, jnp.float32)

        def initbody(j, _):
            for r in range(cpt):
                acc[r][pl.ds(j * 16, 16)] = neg
            return 0

        lax.fori_loop(0, npad // 16, initbody, 0)

        def chunkbody(ch, _):
            pltpu.sync_copy(nbr_hbm.at[pl.ds(ch * qc * KNN, qc * KNN)],
                            nbr_v)

            def gbody(g, _):
                qb = g * 16
                xq = [xv[r][pl.ds(ch * qc + qb, 16)] for r in range(cpt)]
                for j in range(16):
                    idx = nbr_v[pl.ds((qb + j) * KNN, KNN)]
                    for r in range(cpt):
                        vals = jnp.full((16,), xq[r][j], jnp.float32)
                        cur = plsc.load_gather(acc[r], [idx])
                        plsc.store_scatter(acc[r], [idx],
                                           jnp.maximum(cur, vals))
                return 0

            lax.fori_loop(0, qc // 16, gbody, 0)
            return 0

        lax.fori_loop(0, nqc, chunkbody, 0)

        for r in range(cpt):
            pltpu.sync_copy(acc[r], out_hbm.at[c0 + r])

    return sc_scatter


def _lin_bn_body(pt_ref, w_ref, b_ref, g_ref, be_ref, y_ref, *, n, npad):
    nck = npad // CK
    w = w_ref[...]

    def p1(c, s):
        p = pt_ref[:, pl.ds(c * CK, CK)]
        p = jnp.where(p == -jnp.inf, 0.0, p)
        y = lax.dot_general(p, w, (((0,), (1,)), ((), ())),
                            preferred_element_type=jnp.float32)
        y = y + b_ref[...]
        y_ref[pl.ds(c * CK, CK), :] = y
        gidx = lax.broadcasted_iota(jnp.int32, (CK, 1), 0) + c * CK
        return s + jnp.sum(jnp.where(gidx < n, y, 0.0), axis=0,
                           keepdims=True)

    s = lax.fori_loop(0, nck, p1, jnp.zeros((1, 128), jnp.float32))
    mean = s * (1.0 / n)

    def p2(c, v):
        y = y_ref[pl.ds(c * CK, CK), :]
        dev = y - mean
        gidx = lax.broadcasted_iota(jnp.int32, (CK, 1), 0) + c * CK
        return v + jnp.sum(jnp.where(gidx < n, dev * dev, 0.0), axis=0,
                           keepdims=True)

    var = lax.fori_loop(0, nck, p2, jnp.zeros((1, 128), jnp.float32))
    var = var * (1.0 / n)
    scale = g_ref[...] * lax.rsqrt(var + 1e-5)
    shift = be_ref[...] - mean * scale

    def p3(c, _):
        y = y_ref[pl.ds(c * CK, CK), :]
        y_ref[pl.ds(c * CK, CK), :] = jnp.maximum(y * scale + shift, 0.0)
        return 0

    lax.fori_loop(0, nck, p3, 0)


def _lin_bn_relu(pool_t, w, b, gamma, beta, n, npad):
    return pl.pallas_call(
        functools.partial(_lin_bn_body, n=n, npad=npad),
        out_shape=jax.ShapeDtypeStruct((npad, 128), jnp.float32),
    )(pool_t, w, b, gamma, beta)


def kernel(x, batch, W, b, gamma, beta):
    n, din = x.shape
    batch = batch.astype(jnp.int32)
    npad = ((n + CK - 1) // CK) * CK
    nblk = n // RB

    bounds = jnp.searchsorted(batch, jnp.arange(NBATCH + 1, dtype=jnp.int32))
    r0s = jnp.arange(nblk, dtype=jnp.int32) * RB
    lo = bounds[batch[r0s]]
    hi = bounds[batch[r0s + RB - 1] + 1]
    win = jnp.stack([lo // CK, (hi + CK - 1) // CK], axis=1).astype(jnp.int32)

    xpad = jnp.pad(x, ((0, npad - n), (0, 0)))
    bcol = jnp.pad(batch, (0, npad - n), constant_values=-1).reshape(1, npad)
    brow = batch.reshape(n, 1)

    nbr = _knn_topk(x, xpad, brow, bcol, win, n, npad, din)

    xt = x.T
    pool_t = _make_scatter_max(n, npad, din, qc=2000)(nbr.reshape(-1), xt)

    y = _lin_bn_relu(pool_t, W, b.reshape(1, -1), gamma.reshape(1, -1),
                     beta.reshape(1, -1), n, npad)
    return y[:n]
